# baseline (speedup 1.0000x reference)
import jax
import jax.numpy as jnp
from jax.experimental import pallas as pl
from jax.experimental.pallas import tpu as pltpu

_NLAT_IN = 128
_NLON_IN = 256
_NLAT_OUT = 64
_NLON_OUT = 128
_K = 3


def _make_body(BC):
    def _body(offs_ref, la_ref, r_ref, m0_ref, vals_ref, x_ref, out_ref):
        t = pl.program_id(0)
        for k in range(_K):
            def step(e, acc):
                row = x_ref[la_ref[e], r_ref[e], :, :]
                win = pltpu.roll(row, m0_ref[e], axis=0)
                return acc + vals_ref[e] * win

            acc = jax.lax.fori_loop(
                offs_ref[_K * t + k],
                offs_ref[_K * t + k + 1],
                step,
                jnp.zeros((_NLON_OUT, BC), jnp.float32),
            )
            out_ref[0, k, :, :] = acc

    return _body


def kernel(x, psi_vals, psi_idx):
    B, C = x.shape[0], x.shape[1]
    BC = B * C

    xrev = x.reshape(BC, _NLAT_IN, _NLON_OUT, 2)[:, :, ::-1, :].transpose(1, 3, 2, 0)

    kk = psi_idx[0].astype(jnp.int32)
    tt = psi_idx[1].astype(jnp.int32)
    cc = psi_idx[2].astype(jnp.int32)
    la = cc // _NLON_IN
    lo = cc - la * _NLON_IN
    r = lo & 1
    m0 = lo >> 1

    seg = tt * _K + kk
    offs = jnp.searchsorted(
        seg, jnp.arange(_NLAT_OUT * _K + 1, dtype=jnp.int32), side='left'
    ).astype(jnp.int32)

    grid_spec = pltpu.PrefetchScalarGridSpec(
        num_scalar_prefetch=5,
        grid=(_NLAT_OUT,),
        in_specs=[
            pl.BlockSpec(
                (_NLAT_IN, 2, _NLON_OUT, BC),
                lambda t, *_: (0, 0, 0, 0),
            ),
        ],
        out_specs=pl.BlockSpec(
            (1, _K, _NLON_OUT, BC), lambda t, *_: (t, 0, 0, 0)
        ),
    )
    out = pl.pallas_call(
        _make_body(BC),
        grid_spec=grid_spec,
        out_shape=jax.ShapeDtypeStruct((_NLAT_OUT, _K, _NLON_OUT, BC), jnp.float32),
    )(offs, la, r, m0, psi_vals, xrev)

    return out.transpose(3, 1, 0, 2).reshape(B, C, _K, _NLAT_OUT, _NLON_OUT)
